# baseline (speedup 1.0000x reference)
import functools

import jax
import jax.numpy as jnp
import numpy as np
from jax.experimental import pallas as pl
from jax.experimental.pallas import tpu as pltpu

N = 10000
E = 160000
D_IN = 128
D_H = 256
HEADS = 8
D_HEAD = D_H // HEADS
HOPS = 3
ALPHA = 0.1
NCLS = 40
D_FF = 512
NEG = 0.2

NPAD = 10240


def _matmul_kern(x_ref, w_ref, o_ref):
    o_ref[...] = jnp.dot(x_ref[...], w_ref[...],
                         preferred_element_type=jnp.float32)


def _mm(x, w, block_m=1024):
    m, k = x.shape
    k2, n = w.shape
    assert k == k2
    grid = (m // block_m,)
    return pl.pallas_call(
        _matmul_kern,
        grid=grid,
        in_specs=[
            pl.BlockSpec((block_m, k), lambda i: (i, 0)),
            pl.BlockSpec((k, n), lambda i: (0, 0)),
        ],
        out_specs=pl.BlockSpec((block_m, n), lambda i: (i, 0)),
        out_shape=jax.ShapeDtypeStruct((m, n), jnp.float32),
    )(x, w)


def _layer_norm(x):
    mu = jnp.mean(x, axis=-1, keepdims=True)
    var = jnp.var(x, axis=-1, keepdims=True)
    return (x - mu) / jnp.sqrt(var + 1e-5)


def _segment_softmax(scores, dst, num_nodes):
    m = jax.ops.segment_max(scores, dst, num_segments=num_nodes)
    m = jnp.where(jnp.isfinite(m), m, 0.0)
    ex = jnp.exp(scores - m[dst])
    denom = jax.ops.segment_sum(ex, dst, num_segments=num_nodes)
    return ex / (denom[dst] + 1e-16)


def _gdt_layer(h, src, dst, Wq, Wk, Wv, Wo, Wres, Wf1, Wf2):
    n = h.shape[0]
    q = _mm(h, Wq)[:N].reshape(N, HEADS, D_HEAD)
    k = _mm(h, Wk)[:N].reshape(N, HEADS, D_HEAD)
    v = _mm(h, Wv)[:N].reshape(N, HEADS, D_HEAD)
    scores = jnp.sum(q[dst] * k[src], axis=-1) / np.sqrt(D_HEAD)
    scores = jax.nn.leaky_relu(scores, NEG)
    att = _segment_softmax(scores, dst, N)
    feat = v
    for _ in range(HOPS):
        msg = jax.ops.segment_sum(att[:, :, None] * feat[src], dst,
                                  num_segments=N)
        feat = ALPHA * v + (1.0 - ALPHA) * msg
    feat = jnp.pad(feat.reshape(N, D_H), ((0, NPAD - N), (0, 0)))
    out = _mm(feat, Wo)
    res = _mm(h, Wres) if Wres is not None else h
    h1 = _layer_norm(res[:NPAD] + out)
    ff = _mm(jax.nn.relu(_mm(h1, Wf1)), Wf2)
    return _layer_norm(h1 + ff)


def kernel(inputs, edge_index, Wq0, Wk0, Wv0, Wo0, Wres0, Wf10, Wf20,
           Wq1, Wk1, Wv1, Wo1, Wf11, Wf21, Wc, bc):
    loop = jnp.arange(N, dtype=edge_index.dtype)
    src = jnp.concatenate([edge_index[0], loop])
    dst = jnp.concatenate([edge_index[1], loop])
    h0 = jnp.pad(inputs, ((0, NPAD - N), (0, 0)))
    h = _gdt_layer(h0, src, dst, Wq0, Wk0, Wv0, Wo0, Wres0, Wf10, Wf20)
    h = _gdt_layer(h, src, dst, Wq1, Wk1, Wv1, Wo1, None, Wf11, Wf21)
    logits = _mm(h, jnp.pad(Wc, ((0, 0), (0, 128 - NCLS))))[:N, :NCLS] + bc
    return logits
